# Initial kernel scaffold; baseline (speedup 1.0000x reference)
import jax
import jax.numpy as jnp
from jax.experimental import pallas as pl


def kernel(x, emb_table, W_lin, b_lin):
    raise NotImplementedError("write your pallas kernel here")


# speedup vs baseline: 75.7308x; 75.7308x over previous
import functools

import jax
import jax.numpy as jnp
from jax import lax
from jax.experimental import pallas as pl
from jax.experimental.pallas import tpu as pltpu
from jax.experimental.pallas import tpu_sc as plsc

NUM_FIELDS = 26
NUM_FACTOR = 16
BATCH = 16384
FIELD_DIM = 40000
NUM_ROWS = NUM_FIELDS * FIELD_DIM

NC, NS, L = 2, 16, 16
NW = NC * NS
SPW = BATCH // NW
CHUNK = 128
IDX_CH = SPW * NUM_FIELDS // CHUNK
GROUPS = SPW // L

TC_BLK = 16384
TC_GRID = -(-NUM_ROWS // TC_BLK)


def _score_block(tbl_ref, wt_ref, out_ref):
    b = pl.program_id(0)
    c0 = b * TC_BLK
    f0 = c0 // FIELD_DIM
    f1 = jnp.minimum(f0 + 1, NUM_FIELDS - 1)
    boundary = (f0 + 1) * FIELD_DIM - c0
    wt = wt_ref[...]
    fio = lax.broadcasted_iota(jnp.int32, (1, NUM_FIELDS), 1)
    w0 = jnp.sum(wt * (fio == f0), axis=1, keepdims=True)
    w1 = jnp.sum(wt * (fio == f1), axis=1, keepdims=True)
    col = lax.broadcasted_iota(jnp.int32, (1, TC_BLK), 1)
    wsel = jnp.where(col < boundary, w0, w1)
    out_ref[...] = jnp.sum(tbl_ref[...] * wsel, axis=0)


_tc_score = pl.pallas_call(
    _score_block,
    grid=(TC_GRID,),
    in_specs=[
        pl.BlockSpec((NUM_FACTOR, TC_BLK), lambda b: (0, b)),
        pl.BlockSpec((NUM_FACTOR, NUM_FIELDS), lambda b: (0, 0)),
    ],
    out_specs=pl.BlockSpec((TC_BLK,), lambda b: (b,)),
    out_shape=jax.ShapeDtypeStruct((NUM_ROWS,), jnp.float32),
)


_mesh = plsc.VectorSubcoreMesh(core_axis_name="c", subcore_axis_name="s")


@functools.partial(
    pl.kernel,
    mesh=_mesh,
    compiler_params=pltpu.CompilerParams(use_tc_tiling_on_sc=False),
    out_type=jax.ShapeDtypeStruct((BATCH,), jnp.float32),
    scratch_types=[
        pltpu.VMEM((IDX_CH, CHUNK), jnp.int32),
        pltpu.VMEM((IDX_CH * CHUNK,), jnp.float32),
        pltpu.VMEM((SPW,), jnp.float32),
        pltpu.VMEM_SHARED((NUM_ROWS,), jnp.float32),
        pltpu.SemaphoreType.DMA,
    ],
)
def _sc_gather(idx_hbm, scored_hbm, out_hbm,
               idx_v, gbuf, out_v, scored_sh, sem):
    cid = lax.axis_index("c")
    sid = lax.axis_index("s")
    wid = sid * NC + cid
    pltpu.sync_copy(idx_hbm.at[wid], idx_v)

    @pl.when(sid == 0)
    def _stage():
        pltpu.sync_copy(scored_hbm, scored_sh)

    plsc.subcore_barrier()

    copies = [
        pltpu.async_copy(scored_sh.at[idx_v.at[c]],
                         gbuf.at[pl.ds(c * CHUNK, CHUNK)], sem)
        for c in range(IDX_CH)
    ]
    for c in copies:
        c.wait()

    def group_body(g, tok):
        base = g * (L * NUM_FIELDS)
        acc = gbuf[pl.ds(base, L)]
        for f in range(1, NUM_FIELDS):
            acc = acc + gbuf[pl.ds(base + f * L, L)]
        out_v[pl.ds(g * L, L)] = acc
        return tok

    lax.fori_loop(0, GROUPS, group_body, 0)
    pltpu.sync_copy(out_v, out_hbm.at[pl.ds(wid * SPW, SPW)])


def kernel(x, emb_table, W_lin, b_lin):
    table_t = emb_table.T
    w_t = W_lin.reshape(NUM_FIELDS, NUM_FACTOR).T
    scored = _tc_score(table_t, w_t)
    offsets = jnp.arange(NUM_FIELDS, dtype=jnp.int32) * FIELD_DIM
    idx = (x + offsets[None, :])
    idx = idx.reshape(NW, GROUPS, L, NUM_FIELDS).transpose(0, 1, 3, 2)
    scores = _sc_gather(idx.reshape(NW, IDX_CH, CHUNK), scored)
    return scores + b_lin[0]


# speedup vs baseline: 76.7674x; 1.0137x over previous
import functools

import jax
import jax.numpy as jnp
from jax import lax
from jax.experimental import pallas as pl
from jax.experimental.pallas import tpu as pltpu
from jax.experimental.pallas import tpu_sc as plsc

NUM_FIELDS = 26
NUM_FACTOR = 16
BATCH = 16384
FIELD_DIM = 40000
NUM_ROWS = NUM_FIELDS * FIELD_DIM

NC, NS, L = 2, 16, 16
NW = NC * NS
SPW = BATCH // NW
CHUNK = 128
CPF = SPW // CHUNK
GROUPS = SPW // L

TC_BLK = 16384
TC_GRID = -(-NUM_ROWS // TC_BLK)


def _score_block(tbl_ref, wt_ref, out_ref):
    b = pl.program_id(0)
    c0 = b * TC_BLK
    f0 = c0 // FIELD_DIM
    f1 = jnp.minimum(f0 + 1, NUM_FIELDS - 1)
    boundary = (f0 + 1) * FIELD_DIM - c0
    wt = wt_ref[...]
    fio = lax.broadcasted_iota(jnp.int32, (1, NUM_FIELDS), 1)
    w0 = jnp.sum(wt * (fio == f0), axis=1, keepdims=True)
    w1 = jnp.sum(wt * (fio == f1), axis=1, keepdims=True)
    w01 = jnp.concatenate([w0, w1], axis=1)
    mm = lax.dot_general(w01, tbl_ref[...],
                         (((0,), (0,)), ((), ())),
                         preferred_element_type=jnp.float32,
                         precision=lax.Precision.HIGHEST)
    col = lax.iota(jnp.int32, TC_BLK)
    out_ref[...] = jnp.where(col < boundary, mm[0, :], mm[1, :])


_tc_score = pl.pallas_call(
    _score_block,
    grid=(TC_GRID,),
    in_specs=[
        pl.BlockSpec((NUM_FACTOR, TC_BLK), lambda b: (0, b)),
        pl.BlockSpec((NUM_FACTOR, NUM_FIELDS), lambda b: (0, 0)),
    ],
    out_specs=pl.BlockSpec((TC_BLK,), lambda b: (b,)),
    out_shape=jax.ShapeDtypeStruct((NUM_ROWS,), jnp.float32),
)


_mesh = plsc.VectorSubcoreMesh(core_axis_name="c", subcore_axis_name="s")


@functools.partial(
    pl.kernel,
    mesh=_mesh,
    compiler_params=pltpu.CompilerParams(use_tc_tiling_on_sc=False),
    out_type=jax.ShapeDtypeStruct((BATCH,), jnp.float32),
    scratch_types=[
        pltpu.VMEM((NUM_FIELDS, SPW), jnp.int32),
        pltpu.VMEM((NUM_FIELDS * SPW,), jnp.float32),
        pltpu.VMEM((SPW,), jnp.float32),
        pltpu.VMEM_SHARED((NUM_ROWS,), jnp.float32),
        pltpu.SemaphoreType.DMA,
    ],
)
def _sc_gather(idx_hbm, scored_hbm, out_hbm,
               idx_v, gbuf, out_v, scored_sh, sem):
    cid = lax.axis_index("c")
    sid = lax.axis_index("s")
    wid = sid * NC + cid
    pltpu.sync_copy(idx_hbm.at[:, pl.ds(wid * SPW, SPW)], idx_v)

    @pl.when(sid == 0)
    def _stage():
        pltpu.sync_copy(scored_hbm, scored_sh)

    plsc.subcore_barrier()

    copies = [
        pltpu.async_copy(scored_sh.at[idx_v.at[f, pl.ds(k * CHUNK, CHUNK)]],
                         gbuf.at[pl.ds(f * SPW + k * CHUNK, CHUNK)], sem)
        for f in range(NUM_FIELDS)
        for k in range(CPF)
    ]
    for c in copies:
        c.wait()

    def group_body(g, tok):
        base = g * L
        acc = gbuf[pl.ds(base, L)]
        for f in range(1, NUM_FIELDS):
            acc = acc + gbuf[pl.ds(f * SPW + base, L)]
        out_v[pl.ds(base, L)] = acc
        return tok

    lax.fori_loop(0, GROUPS, group_body, 0)
    pltpu.sync_copy(out_v, out_hbm.at[pl.ds(wid * SPW, SPW)])


def kernel(x, emb_table, W_lin, b_lin):
    table_t = emb_table.T
    w_t = W_lin.reshape(NUM_FIELDS, NUM_FACTOR).T
    scored = _tc_score(table_t, w_t)
    offsets = jnp.arange(NUM_FIELDS, dtype=jnp.int32) * FIELD_DIM
    idx_t = x.T + offsets[:, None]
    scores = _sc_gather(idx_t, scored)
    return scores + b_lin[0]


# speedup vs baseline: 118.3434x; 1.5416x over previous
import functools

import jax
import jax.numpy as jnp
from jax import lax
from jax.experimental import pallas as pl
from jax.experimental.pallas import tpu as pltpu
from jax.experimental.pallas import tpu_sc as plsc

NUM_FIELDS = 26
NUM_FACTOR = 16
BATCH = 16384
FIELD_DIM = 40000
NUM_ROWS = NUM_FIELDS * FIELD_DIM

NC, NS, L = 2, 16, 16
NW = NC * NS
SPW = BATCH // NW
CHUNK = 128
CPF = SPW // CHUNK
GROUPS = SPW // L

TC_BLK = 32768
TC_GRID = -(-NUM_ROWS // TC_BLK)


def _score_block(tbl_ref, wt_ref, out_ref):
    b = pl.program_id(0)
    c0 = b * TC_BLK
    f0 = c0 // FIELD_DIM
    f1 = jnp.minimum(f0 + 1, NUM_FIELDS - 1)
    boundary = (f0 + 1) * FIELD_DIM - c0
    wt = wt_ref[...]
    fio = lax.broadcasted_iota(jnp.int32, (1, NUM_FIELDS), 1)
    w0 = jnp.sum(wt * (fio == f0), axis=1, keepdims=True)
    w1 = jnp.sum(wt * (fio == f1), axis=1, keepdims=True)
    col = lax.broadcasted_iota(jnp.int32, (1, TC_BLK), 1)
    wsel = jnp.where(col < boundary, w0, w1)
    out_ref[...] = jnp.sum(tbl_ref[...] * wsel, axis=0)


_tc_score = pl.pallas_call(
    _score_block,
    grid=(TC_GRID,),
    in_specs=[
        pl.BlockSpec((NUM_FACTOR, TC_BLK), lambda b: (0, b)),
        pl.BlockSpec((NUM_FACTOR, NUM_FIELDS), lambda b: (0, 0)),
    ],
    out_specs=pl.BlockSpec((TC_BLK,), lambda b: (b,)),
    out_shape=jax.ShapeDtypeStruct((NUM_ROWS,), jnp.float32),
)


_mesh = plsc.VectorSubcoreMesh(core_axis_name="c", subcore_axis_name="s")


@functools.partial(
    pl.kernel,
    mesh=_mesh,
    compiler_params=pltpu.CompilerParams(use_tc_tiling_on_sc=False),
    out_type=jax.ShapeDtypeStruct((BATCH,), jnp.float32),
    scratch_types=[
        pltpu.VMEM((NUM_FIELDS, SPW), jnp.int32),
        pltpu.VMEM((NUM_FIELDS * SPW,), jnp.float32),
        pltpu.VMEM((SPW,), jnp.float32),
        pltpu.VMEM_SHARED((NUM_ROWS,), jnp.float32),
        pltpu.SemaphoreType.DMA,
    ],
)
def _sc_gather(idx_hbm, scored_hbm, out_hbm,
               idx_v, gbuf, out_v, scored_sh, sem):
    cid = lax.axis_index("c")
    sid = lax.axis_index("s")
    wid = sid * NC + cid
    pltpu.sync_copy(idx_hbm.at[:, pl.ds(wid * SPW, SPW)], idx_v)

    @pl.when(sid == 0)
    def _stage():
        pltpu.sync_copy(scored_hbm, scored_sh)

    plsc.subcore_barrier()

    copies = [
        pltpu.async_copy(scored_sh.at[idx_v.at[f, pl.ds(k * CHUNK, CHUNK)]],
                         gbuf.at[pl.ds(f * SPW + k * CHUNK, CHUNK)], sem)
        for f in range(NUM_FIELDS)
        for k in range(CPF)
    ]
    for c in copies:
        c.wait()

    def group_body(g, tok):
        base = g * L
        acc = gbuf[pl.ds(base, L)]
        for f in range(1, NUM_FIELDS):
            acc = acc + gbuf[pl.ds(f * SPW + base, L)]
        out_v[pl.ds(base, L)] = acc
        return tok

    lax.fori_loop(0, GROUPS, group_body, 0)
    pltpu.sync_copy(out_v, out_hbm.at[pl.ds(wid * SPW, SPW)])


def kernel(x, emb_table, W_lin, b_lin):
    table_t = emb_table.T
    w_t = W_lin.reshape(NUM_FIELDS, NUM_FACTOR).T
    scored = _tc_score(table_t, w_t)
    offsets = jnp.arange(NUM_FIELDS, dtype=jnp.int32) * FIELD_DIM
    idx_t = x.T + offsets[:, None]
    scores = _sc_gather(idx_t, scored)
    return scores + b_lin[0]


# speedup vs baseline: 126.9644x; 1.0728x over previous
import functools

import jax
import jax.numpy as jnp
from jax import lax
from jax.experimental import pallas as pl
from jax.experimental.pallas import tpu as pltpu
from jax.experimental.pallas import tpu_sc as plsc

NUM_FIELDS = 26
NUM_FACTOR = 16
BATCH = 16384
FIELD_DIM = 40000
NUM_ROWS = NUM_FIELDS * FIELD_DIM

NC, NS, L = 2, 16, 16
NW = NC * NS
SPW = BATCH // NW
CHUNK = 128
CPF = SPW // CHUNK
GROUPS = SPW // L

TC_BLK = 65536
TC_GRID = -(-NUM_ROWS // TC_BLK)
SH_SLICE = NUM_ROWS // NS


def _score_block(tbl_ref, wt_ref, out_ref):
    b = pl.program_id(0)
    c0 = b * TC_BLK
    f0 = c0 // FIELD_DIM
    f1 = jnp.minimum(f0 + 1, NUM_FIELDS - 1)
    boundary = (f0 + 1) * FIELD_DIM - c0
    wt = wt_ref[...]
    fio = lax.broadcasted_iota(jnp.int32, (1, NUM_FIELDS), 1)
    w0 = jnp.sum(wt * (fio == f0), axis=1, keepdims=True)
    w1 = jnp.sum(wt * (fio == f1), axis=1, keepdims=True)
    col = lax.broadcasted_iota(jnp.int32, (1, TC_BLK), 1)
    wsel = jnp.where(col < boundary, w0, w1)
    out_ref[...] = jnp.sum(tbl_ref[...] * wsel, axis=0)


_tc_score = pl.pallas_call(
    _score_block,
    grid=(TC_GRID,),
    in_specs=[
        pl.BlockSpec((NUM_FACTOR, TC_BLK), lambda b: (0, b)),
        pl.BlockSpec((NUM_FACTOR, NUM_FIELDS), lambda b: (0, 0)),
    ],
    out_specs=pl.BlockSpec((TC_BLK,), lambda b: (b,)),
    out_shape=jax.ShapeDtypeStruct((NUM_ROWS,), jnp.float32),
)


_mesh = plsc.VectorSubcoreMesh(core_axis_name="c", subcore_axis_name="s")


@functools.partial(
    pl.kernel,
    mesh=_mesh,
    compiler_params=pltpu.CompilerParams(use_tc_tiling_on_sc=False),
    out_type=jax.ShapeDtypeStruct((BATCH,), jnp.float32),
    scratch_types=[
        pltpu.VMEM((NUM_FIELDS, SPW), jnp.int32),
        pltpu.VMEM((NUM_FIELDS * SPW,), jnp.float32),
        pltpu.VMEM((SPW,), jnp.float32),
        pltpu.VMEM((L,), jnp.float32),
        pltpu.VMEM_SHARED((NUM_ROWS,), jnp.float32),
        pltpu.SemaphoreType.DMA,
    ],
)
def _sc_gather(idx_hbm, scored_hbm, bias_hbm, out_hbm,
               idx_v, gbuf, out_v, bias_v, scored_sh, sem):
    cid = lax.axis_index("c")
    sid = lax.axis_index("s")
    wid = sid * NC + cid
    pltpu.sync_copy(scored_hbm.at[pl.ds(sid * SH_SLICE, SH_SLICE)],
                    scored_sh.at[pl.ds(sid * SH_SLICE, SH_SLICE)])
    pltpu.sync_copy(idx_hbm.at[:, pl.ds(wid * SPW, SPW)], idx_v)
    pltpu.sync_copy(bias_hbm, bias_v)
    plsc.subcore_barrier()

    copies = [
        pltpu.async_copy(scored_sh.at[idx_v.at[f, pl.ds(k * CHUNK, CHUNK)]],
                         gbuf.at[pl.ds(f * SPW + k * CHUNK, CHUNK)], sem)
        for f in range(NUM_FIELDS)
        for k in range(CPF)
    ]
    for c in copies:
        c.wait()

    def group_body(g, tok):
        base = g * L
        acc = bias_v[...] + gbuf[pl.ds(base, L)]
        for f in range(1, NUM_FIELDS):
            acc = acc + gbuf[pl.ds(f * SPW + base, L)]
        out_v[pl.ds(base, L)] = acc
        return tok

    lax.fori_loop(0, GROUPS, group_body, 0)
    pltpu.sync_copy(out_v, out_hbm.at[pl.ds(wid * SPW, SPW)])


def kernel(x, emb_table, W_lin, b_lin):
    table_t = emb_table.T
    w_t = W_lin.reshape(NUM_FIELDS, NUM_FACTOR).T
    scored = _tc_score(table_t, w_t)
    offsets = jnp.arange(NUM_FIELDS, dtype=jnp.int32) * FIELD_DIM
    idx_t = x.T + offsets[:, None]
    bias_vec = jnp.full((L,), b_lin[0], jnp.float32)
    return _sc_gather(idx_t, scored, bias_vec)
